# baseline (speedup 1.0000x reference)
import jax
import jax.numpy as jnp
from jax import lax
from jax.experimental import pallas as pl
from jax.experimental.pallas import tpu as pltpu
from jax.experimental.pallas import tpu_sc as plsc

_B = 16384
_D = 512
_V = 1000
_NC = 2
_NS = 16
_NW = _NC * _NS
_BPW = _B // _NW
_CH = 32
_NCHUNK = _BPW // _CH
_NBUF = 7
_LAG = 2


def _table_body(emb_ref, w1_ref, b1_ref, w2_ref, b2_ref, out_ref):
    p = jnp.dot(emb_ref[...], w1_ref[...], preferred_element_type=jnp.float32)
    p = p + b1_ref[...]
    p = p * jax.nn.sigmoid(p)
    q = jnp.dot(p, w2_ref[...], preferred_element_type=jnp.float32)
    q = q + b2_ref[...]
    out_ref[...] = q * jax.nn.sigmoid(q)


_TGRID = 5
_TBLK = _V // _TGRID


def _build_table(embeddings, W1, b1, W2, b2):
    return pl.pallas_call(
        _table_body,
        grid=(_TGRID,),
        out_shape=jax.ShapeDtypeStruct((_V, _D), jnp.float32),
        in_specs=[
            pl.BlockSpec((_TBLK, _V), lambda i: (i, 0)),
            pl.BlockSpec((_V, _D), lambda i: (0, 0)),
            pl.BlockSpec((1, _D), lambda i: (0, 0)),
            pl.BlockSpec((_D, _D), lambda i: (0, 0)),
            pl.BlockSpec((1, _D), lambda i: (0, 0)),
        ],
        out_specs=pl.BlockSpec((_TBLK, _D), lambda i: (i, 0)),
    )(embeddings, W1, b1.reshape(1, _D), W2, b2.reshape(1, _D))


def _gather_body(table_hbm, idx_hbm, out_hbm, idx_v,
                 rows0, rows1, rows2, rows3, rows4, rows5, rows6,
                 gsem0, gsem1, gsem2, gsem3, gsem4, gsem5, gsem6,
                 osem0, osem1, osem2, osem3, osem4, osem5, osem6):
    wid = lax.axis_index("s") * _NC + lax.axis_index("c")
    base = wid * _BPW
    pltpu.sync_copy(idx_hbm.at[pl.ds(base, _BPW)], idx_v)
    bufs = (rows0, rows1, rows2, rows3, rows4, rows5, rows6)
    gsems = (gsem0, gsem1, gsem2, gsem3, gsem4, gsem5, gsem6)
    osems = (osem0, osem1, osem2, osem3, osem4, osem5, osem6)
    depth = _NBUF - _LAG
    gathers = [None] * _NBUF
    outs = [None] * _NCHUNK
    for c in range(min(depth, _NCHUNK)):
        gathers[c % _NBUF] = pltpu.async_copy(
            table_hbm.at[idx_v.at[pl.ds(c * _CH, _CH)]],
            bufs[c % _NBUF], gsems[c % _NBUF])
    for c in range(_NCHUNK):
        b = c % _NBUF
        gathers[b].wait()
        outs[c] = pltpu.async_copy(
            bufs[b], out_hbm.at[pl.ds(base + c * _CH, _CH)], osems[b])
        nxt = c + depth
        if nxt < _NCHUNK:
            nb = nxt % _NBUF
            prev = nxt - _NBUF
            if prev >= 0:
                outs[prev].wait()
            gathers[nb] = pltpu.async_copy(
                table_hbm.at[idx_v.at[pl.ds(nxt * _CH, _CH)]],
                bufs[nb], gsems[nb])
    for c in range(_NCHUNK - _NBUF, _NCHUNK):
        if c >= 0:
            outs[c].wait()


_gather_call = pl.kernel(
    _gather_body,
    out_type=jax.ShapeDtypeStruct((_B, _D), jnp.float32),
    mesh=plsc.VectorSubcoreMesh(core_axis_name="c", subcore_axis_name="s"),
    scratch_types=[
        pltpu.VMEM((_BPW,), jnp.int32),
        pltpu.VMEM((_CH, _D), jnp.float32),
        pltpu.VMEM((_CH, _D), jnp.float32),
        pltpu.VMEM((_CH, _D), jnp.float32),
        pltpu.VMEM((_CH, _D), jnp.float32),
        pltpu.VMEM((_CH, _D), jnp.float32),
        pltpu.VMEM((_CH, _D), jnp.float32),
        pltpu.VMEM((_CH, _D), jnp.float32),
        pltpu.SemaphoreType.DMA,
        pltpu.SemaphoreType.DMA,
        pltpu.SemaphoreType.DMA,
        pltpu.SemaphoreType.DMA,
        pltpu.SemaphoreType.DMA,
        pltpu.SemaphoreType.DMA,
        pltpu.SemaphoreType.DMA,
        pltpu.SemaphoreType.DMA,
        pltpu.SemaphoreType.DMA,
        pltpu.SemaphoreType.DMA,
        pltpu.SemaphoreType.DMA,
        pltpu.SemaphoreType.DMA,
        pltpu.SemaphoreType.DMA,
        pltpu.SemaphoreType.DMA,
    ],
)


def kernel(step, embeddings, W1, b1, W2, b2):
    table = _build_table(embeddings, W1, b1, W2, b2)
    idx = step.astype(jnp.int32)
    out = _gather_call(table, idx)
    return out[None]


# speedup vs baseline: 1.0138x; 1.0138x over previous
import jax
import jax.numpy as jnp
from jax import lax
from jax.experimental import pallas as pl
from jax.experimental.pallas import tpu as pltpu
from jax.experimental.pallas import tpu_sc as plsc

_B = 16384
_D = 512
_V = 1000
_NC = 2
_NS = 16
_NW = _NC * _NS
_BPW = _B // _NW
_CH = 64
_NCHUNK = _BPW // _CH
_NBUF = 3
_LAG = 1


def _table_body(emb_ref, w1_ref, b1_ref, w2_ref, b2_ref, out_ref):
    p = jnp.dot(emb_ref[...], w1_ref[...], preferred_element_type=jnp.float32)
    p = p + b1_ref[...]
    p = p * jax.nn.sigmoid(p)
    q = jnp.dot(p, w2_ref[...], preferred_element_type=jnp.float32)
    q = q + b2_ref[...]
    out_ref[...] = q * jax.nn.sigmoid(q)


def _build_table(embeddings, W1, b1, W2, b2):
    return pl.pallas_call(
        _table_body,
        out_shape=jax.ShapeDtypeStruct((_V, _D), jnp.float32),
        in_specs=[
            pl.BlockSpec(memory_space=pltpu.VMEM),
            pl.BlockSpec(memory_space=pltpu.VMEM),
            pl.BlockSpec(memory_space=pltpu.VMEM),
            pl.BlockSpec(memory_space=pltpu.VMEM),
            pl.BlockSpec(memory_space=pltpu.VMEM),
        ],
        out_specs=pl.BlockSpec(memory_space=pltpu.VMEM),
    )(embeddings, W1, b1.reshape(1, _D), W2, b2.reshape(1, _D))


def _gather_body(table_hbm, idx_hbm, out_hbm, idx_v,
                 rows0, rows1, rows2,
                 gsem0, gsem1, gsem2, osem0, osem1, osem2):
    wid = lax.axis_index("s") * _NC + lax.axis_index("c")
    base = wid * _BPW
    pltpu.sync_copy(idx_hbm.at[pl.ds(base, _BPW)], idx_v)
    bufs = (rows0, rows1, rows2)
    gsems = (gsem0, gsem1, gsem2)
    osems = (osem0, osem1, osem2)
    depth = _NBUF - _LAG
    gathers = [None] * _NBUF
    outs = [None] * _NCHUNK
    for c in range(min(depth, _NCHUNK)):
        gathers[c % _NBUF] = pltpu.async_copy(
            table_hbm.at[idx_v.at[pl.ds(c * _CH, _CH)]],
            bufs[c % _NBUF], gsems[c % _NBUF])
    for c in range(_NCHUNK):
        b = c % _NBUF
        gathers[b].wait()
        outs[c] = pltpu.async_copy(
            bufs[b], out_hbm.at[pl.ds(base + c * _CH, _CH)], osems[b])
        nxt = c + depth
        if nxt < _NCHUNK:
            nb = nxt % _NBUF
            prev = nxt - _NBUF
            if prev >= 0:
                outs[prev].wait()
            gathers[nb] = pltpu.async_copy(
                table_hbm.at[idx_v.at[pl.ds(nxt * _CH, _CH)]],
                bufs[nb], gsems[nb])
    for c in range(_NCHUNK - _NBUF, _NCHUNK):
        if c >= 0:
            outs[c].wait()


_gather_call = pl.kernel(
    _gather_body,
    out_type=jax.ShapeDtypeStruct((_B, _D), jnp.float32),
    mesh=plsc.VectorSubcoreMesh(core_axis_name="c", subcore_axis_name="s"),
    scratch_types=[
        pltpu.VMEM((_BPW,), jnp.int32),
        pltpu.VMEM((_CH, _D), jnp.float32),
        pltpu.VMEM((_CH, _D), jnp.float32),
        pltpu.VMEM((_CH, _D), jnp.float32),
        pltpu.SemaphoreType.DMA,
        pltpu.SemaphoreType.DMA,
        pltpu.SemaphoreType.DMA,
        pltpu.SemaphoreType.DMA,
        pltpu.SemaphoreType.DMA,
        pltpu.SemaphoreType.DMA,
    ],
)


def kernel(step, embeddings, W1, b1, W2, b2):
    table = _build_table(embeddings, W1, b1, W2, b2)
    idx = step.astype(jnp.int32)
    out = _gather_call(table, idx)
    return out[None]


# speedup vs baseline: 1.0254x; 1.0115x over previous
import jax
import jax.numpy as jnp
from jax import lax
from jax.experimental import pallas as pl
from jax.experimental.pallas import tpu as pltpu
from jax.experimental.pallas import tpu_sc as plsc

_B = 16384
_D = 512
_V = 1000
_NC = 2
_NS = 16
_NW = _NC * _NS
_BPW = _B // _NW
_CH = 32
_NCHUNK = _BPW // _CH
_NBUF = 7
_LAG = 2


def _table_body(emb_ref, w1_ref, b1_ref, w2_ref, b2_ref, out_ref):
    p = jnp.dot(emb_ref[...], w1_ref[...], preferred_element_type=jnp.float32)
    p = p + b1_ref[...]
    p = p * jax.nn.sigmoid(p)
    q = jnp.dot(p, w2_ref[...], preferred_element_type=jnp.float32)
    q = q + b2_ref[...]
    out_ref[...] = q * jax.nn.sigmoid(q)


def _build_table(embeddings, W1, b1, W2, b2):
    return pl.pallas_call(
        _table_body,
        out_shape=jax.ShapeDtypeStruct((_V, _D), jnp.float32),
        in_specs=[
            pl.BlockSpec(memory_space=pltpu.VMEM),
            pl.BlockSpec(memory_space=pltpu.VMEM),
            pl.BlockSpec(memory_space=pltpu.VMEM),
            pl.BlockSpec(memory_space=pltpu.VMEM),
            pl.BlockSpec(memory_space=pltpu.VMEM),
        ],
        out_specs=pl.BlockSpec(memory_space=pltpu.VMEM),
    )(embeddings, W1, b1.reshape(1, _D), W2, b2.reshape(1, _D))


def _gather_body(table_hbm, idx_hbm, out_hbm, idx_v,
                 rows0, rows1, rows2, rows3, rows4, rows5, rows6,
                 gsem0, gsem1, gsem2, gsem3, gsem4, gsem5, gsem6,
                 osem0, osem1, osem2, osem3, osem4, osem5, osem6):
    wid = lax.axis_index("s") * _NC + lax.axis_index("c")
    base = wid * _BPW
    pltpu.sync_copy(idx_hbm.at[pl.ds(base, _BPW)], idx_v)
    bufs = (rows0, rows1, rows2, rows3, rows4, rows5, rows6)
    gsems = (gsem0, gsem1, gsem2, gsem3, gsem4, gsem5, gsem6)
    osems = (osem0, osem1, osem2, osem3, osem4, osem5, osem6)
    depth = _NBUF - _LAG
    gathers = [None] * _NBUF
    outs = [None] * _NCHUNK
    for c in range(min(depth, _NCHUNK)):
        gathers[c % _NBUF] = pltpu.async_copy(
            table_hbm.at[idx_v.at[pl.ds(c * _CH, _CH)]],
            bufs[c % _NBUF], gsems[c % _NBUF])
    for c in range(_NCHUNK):
        b = c % _NBUF
        gathers[b].wait()
        outs[c] = pltpu.async_copy(
            bufs[b], out_hbm.at[pl.ds(base + c * _CH, _CH)], osems[b])
        nxt = c + depth
        if nxt < _NCHUNK:
            nb = nxt % _NBUF
            prev = nxt - _NBUF
            if prev >= 0:
                outs[prev].wait()
            gathers[nb] = pltpu.async_copy(
                table_hbm.at[idx_v.at[pl.ds(nxt * _CH, _CH)]],
                bufs[nb], gsems[nb])
    for c in range(_NCHUNK - _NBUF, _NCHUNK):
        if c >= 0:
            outs[c].wait()


_gather_call = pl.kernel(
    _gather_body,
    out_type=jax.ShapeDtypeStruct((_B, _D), jnp.float32),
    mesh=plsc.VectorSubcoreMesh(core_axis_name="c", subcore_axis_name="s"),
    scratch_types=[
        pltpu.VMEM((_BPW,), jnp.int32),
        pltpu.VMEM((_CH, _D), jnp.float32),
        pltpu.VMEM((_CH, _D), jnp.float32),
        pltpu.VMEM((_CH, _D), jnp.float32),
        pltpu.VMEM((_CH, _D), jnp.float32),
        pltpu.VMEM((_CH, _D), jnp.float32),
        pltpu.VMEM((_CH, _D), jnp.float32),
        pltpu.VMEM((_CH, _D), jnp.float32),
        pltpu.SemaphoreType.DMA,
        pltpu.SemaphoreType.DMA,
        pltpu.SemaphoreType.DMA,
        pltpu.SemaphoreType.DMA,
        pltpu.SemaphoreType.DMA,
        pltpu.SemaphoreType.DMA,
        pltpu.SemaphoreType.DMA,
        pltpu.SemaphoreType.DMA,
        pltpu.SemaphoreType.DMA,
        pltpu.SemaphoreType.DMA,
        pltpu.SemaphoreType.DMA,
        pltpu.SemaphoreType.DMA,
        pltpu.SemaphoreType.DMA,
        pltpu.SemaphoreType.DMA,
    ],
)


def kernel(step, embeddings, W1, b1, W2, b2):
    table = _build_table(embeddings, W1, b1, W2, b2)
    idx = step.astype(jnp.int32)
    out = _gather_call(table, idx)
    return out[None]
